# baseline (speedup 1.0000x reference)
import jax
import jax.numpy as jnp
from jax import lax
from jax.experimental import pallas as pl
from jax.experimental.pallas import tpu as pltpu
from jax.experimental.pallas import tpu_sc as plsc

_B, _H, _W = 8, 512, 512
_N = _B * _H * _W
_NEG_RATIO = 3.0
_EPS = 1e-06


_NCORES, _NSUB = 2, 16
_NW = _NCORES * _NSUB
_PER = _N // _NW
_CHUNK = 16384
_NCH = _PER // _CHUNK
_VPG = 8
_LANES = 16

_LOG1P_C = (-1.7414116888e-02, 8.2691420711e-02, -1.9035463582e-01,
            3.1574753796e-01, -4.9737329285e-01, 9.9984770861e-01,
            1.4716138946e-06)


def _sc_body(x_hbm, gt_hbm, m_hbm, stats_out, neg_out,
             xb, gb, mb, ob, sbuf, semx, semg, semm):
    c = lax.axis_index("c")
    s = lax.axis_index("s")
    w = c * _NSUB + s
    base_w = w * _PER

    zero = jnp.zeros((_LANES,), jnp.float32)
    accs = (zero, zero, zero, zero)

    for ch in range(_NCH):
        base = base_w + ch * _CHUNK
        cpx = pltpu.async_copy(x_hbm.at[pl.ds(base, _CHUNK)], xb, semx)
        cpg = pltpu.async_copy(gt_hbm.at[pl.ds(base, _CHUNK)], gb, semg)
        cpm = pltpu.async_copy(m_hbm.at[pl.ds(base, _CHUNK)], mb, semm)
        cpx.wait()
        cpg.wait()
        cpm.wait()

        def _step(i, accs):
            a, b, cc, d = accs
            for j in range(_VPG):
                off = (i * _VPG + j) * _LANES
                xv = xb[pl.ds(off, _LANES)]
                gi = gb[pl.ds(off, _LANES)]
                mi = mb[pl.ds(off, _LANES)]
                gtv = jnp.where(gi > 0, 1.0, 0.0)
                mv = mi.astype(jnp.float32)
                e = jnp.exp(-jnp.abs(xv))
                p = jnp.full((_LANES,), _LOG1P_C[0], jnp.float32)
                for coef in _LOG1P_C[1:]:
                    p = p * e + jnp.float32(coef)
                sp = jnp.maximum(xv, 0.0) + p
                gm = gtv * mv
                lossv = sp - xv * gtv
                negv = sp * (mv - gm)
                a = a + gm
                b = b + mv
                cc = cc + lossv * gm
                d = d + negv
                ob[pl.ds(off, _LANES)] = negv
            return (a, b, cc, d)

        accs = lax.fori_loop(0, _CHUNK // (_VPG * _LANES), _step, accs)
        pltpu.sync_copy(ob, neg_out.at[pl.ds(base, _CHUNK)])

    sbuf[0, :] = accs[0]
    sbuf[1, :] = accs[1]
    sbuf[2, :] = accs[2]
    sbuf[3, :] = accs[3]
    pltpu.sync_copy(sbuf, stats_out.at[w])


def _sc_pass(xf, gtf, mf):
    mesh = plsc.VectorSubcoreMesh(core_axis_name="c", subcore_axis_name="s")
    return pl.kernel(
        _sc_body,
        out_type=[jax.ShapeDtypeStruct((_NW, 4, _LANES), jnp.float32),
                  jax.ShapeDtypeStruct((_N,), jnp.float32)],
        mesh=mesh,
        scratch_types=[
            pltpu.VMEM((_CHUNK,), jnp.float32),
            pltpu.VMEM((_CHUNK,), jnp.int32),
            pltpu.VMEM((_CHUNK,), jnp.int32),
            pltpu.VMEM((_CHUNK,), jnp.float32),
            pltpu.VMEM((4, _LANES), jnp.float32),
            pltpu.SemaphoreType.DMA,
            pltpu.SemaphoreType.DMA,
            pltpu.SemaphoreType.DMA,
        ],
    )(xf, gtf, mf)


_ROWS, _COLS = 2048, 1024
_TILE = 256
_NT = _ROWS // _TILE
_NROUNDS = 16

_S_C1, _S_C2, _S_C3, _S_RELU, _S_KF = range(5)


def _sel_body(k_ref, neg_ref, out_ref, smf, smi):
    r = pl.program_id(0)
    t = pl.program_id(1)

    @pl.when((r == 0) & (t == 0))
    def _init():
        smi[0] = 0
        smf[_S_KF] = k_ref[0].astype(jnp.float32)
        smf[_S_RELU] = 0.0

    @pl.when(r < _NROUNDS)
    def _bisect():
        step = jnp.maximum(jnp.int32(1), jnp.int32(1 << 29) >> (2 * r))

        @pl.when(t == 0)
        def _zero_counts():
            smf[_S_C1] = 0.0
            smf[_S_C2] = 0.0
            smf[_S_C3] = 0.0

        lo = smi[0]
        bits = jax.lax.bitcast_convert_type(neg_ref[...], jnp.int32)
        smf[_S_C1] += jnp.sum((bits >= lo + step).astype(jnp.float32))
        smf[_S_C2] += jnp.sum((bits >= lo + 2 * step).astype(jnp.float32))
        smf[_S_C3] += jnp.sum((bits >= lo + 3 * step).astype(jnp.float32))

        @pl.when(t == _NT - 1)
        def _decide():
            kf = smf[_S_KF]
            jmax = ((smf[_S_C1] >= kf).astype(jnp.int32)
                    + (smf[_S_C2] >= kf).astype(jnp.int32)
                    + (smf[_S_C3] >= kf).astype(jnp.int32))
            smi[0] = lo + jmax * step

    @pl.when(r == _NROUNDS)
    def _final():
        tstar = jax.lax.bitcast_convert_type(smi[0], jnp.float32)
        smf[_S_RELU] += jnp.sum(jnp.maximum(neg_ref[...] - tstar, 0.0))

        @pl.when(t == _NT - 1)
        def _assemble():
            kf = smf[_S_KF]
            out_ref[0] = jnp.where(
                k_ref[0] > 0, smf[_S_RELU] + kf * tstar, 0.0)


def _tc_topk_sum(neg2d, k):
    return pl.pallas_call(
        _sel_body,
        grid=(_NROUNDS + 1, _NT),
        in_specs=[
            pl.BlockSpec(memory_space=pltpu.SMEM),
            pl.BlockSpec((_TILE, _COLS), lambda r, t: (t, 0)),
        ],
        out_specs=pl.BlockSpec(memory_space=pltpu.SMEM),
        out_shape=jax.ShapeDtypeStruct((1,), jnp.float32),
        scratch_shapes=[
            pltpu.SMEM((5,), jnp.float32),
            pltpu.SMEM((1,), jnp.int32),
        ],
        compiler_params=pltpu.CompilerParams(
            dimension_semantics=("arbitrary", "arbitrary")),
    )(k.reshape(1), neg2d)[0]


def kernel(preds, downsample_ratio, gt_shrink, gt_shrink_mask):
    xf = preds.reshape(_N)
    gtf = gt_shrink.reshape(_N)
    mf = gt_shrink_mask.reshape(_N)
    stats, neg = _sc_pass(xf, gtf, mf)

    st = jnp.sum(stats, axis=(0, 2))
    pos_f, mask_f, pos_loss, neg_sum = st[0], st[1], st[2], st[3]
    neg_i = (mask_f - pos_f).astype(jnp.int32)
    cap = (pos_f * _NEG_RATIO).astype(jnp.int32)
    k = jnp.minimum(neg_i, cap)

    neg_top = lax.cond(
        cap < neg_i,
        lambda: _tc_topk_sum(neg.reshape(_ROWS, _COLS), k),
        lambda: neg_sum)

    denom = (pos_f.astype(jnp.int32) + k).astype(jnp.float32) + _EPS
    loss = (pos_loss + neg_top) / denom
    return loss * downsample_ratio
